# baseline (speedup 1.0000x reference)
import jax
import jax.numpy as jnp
from jax import lax
from jax.experimental import pallas as pl
from jax.experimental.pallas import tpu as pltpu
from jax.experimental.pallas import tpu_sc as plsc

N = 256
S = 256
D = 256
NC = 2
NS = 16
NW = NC * NS
IB = 32
JB = 64
NJ = S // JB
JC = 32
STEPS = IB * JB // JC
NBUF = 12
LANES = 16

_ORDER = []
for _q in range(0, STEPS, 4):
    _ORDER += [_q, _q + 2, _q + 1, _q + 3]


def _sc_body(x_hbm, table_hbm, out_hbm, idx_v, adnd_v, bufs, gsems, ssems,
             asem):
    wid = lax.axis_index("s") * NC + lax.axis_index("c")
    i0 = (wid // NJ) * IB
    j0 = (wid % NJ) * JB

    pltpu.sync_copy(x_hbm.at[pl.ds(i0, IB)], idx_v)
    a_copy = pltpu.make_async_copy(x_hbm.at[pl.ds(j0, JB)], adnd_v, asem)
    a_copy.start()

    def gather(p):
        k = _ORDER[p]
        return pltpu.make_async_copy(
            table_hbm.at[idx_v.at[k // 2, pl.ds(j0 + JC * (k % 2), JC)]],
            bufs[p % NBUF], gsems[p % NBUF])

    def store(p):
        k = _ORDER[p]
        dst = (i0 + k // 2) * S + j0 + JC * (k % 2)
        return pltpu.make_async_copy(
            bufs[p % NBUF], out_hbm.at[pl.ds(dst, JC)], ssems[p % NBUF])

    def add_pair(p):
        blist = [bufs[p % NBUF], bufs[(p + 1) % NBUF]]
        jbase = JC * (_ORDER[p] % 2)

        def row_body(r, _):
            a = [adnd_v[jbase + r, pl.ds(c * LANES, LANES)]
                 .astype(jnp.float32) for c in range(D // LANES)]
            for buf in blist:
                for c in range(D // LANES):
                    plsc.addupdate(buf.at[r, pl.ds(c * LANES, LANES)], a[c])
            return 0

        lax.fori_loop(0, JC, row_body, 0)

    for p in range(NBUF):
        gather(p).start()
    a_copy.wait()
    for g in range(STEPS // 2):
        p = 2 * g
        gather(p).wait()
        gather(p + 1).wait()
        add_pair(p)
        store(p).start()
        store(p + 1).start()
        for t in (p, p + 1):
            if t + NBUF < STEPS:
                store(t).wait()
                gather(t + NBUF).start()
    for p in range(STEPS - NBUF, STEPS):
        store(p).wait()


@jax.jit
def _pe_lookup(x, table):
    mesh = plsc.VectorSubcoreMesh(core_axis_name="c", subcore_axis_name="s")
    return pl.kernel(
        _sc_body,
        out_type=jax.ShapeDtypeStruct((N * S, D), jnp.float32),
        mesh=mesh,
        scratch_types=[
            pltpu.VMEM((IB, S), jnp.int32),
            pltpu.VMEM((JB, D), jnp.int32),
            tuple(pltpu.VMEM((JC, D), jnp.float32) for _ in range(NBUF)),
            tuple(pltpu.SemaphoreType.DMA for _ in range(NBUF)),
            tuple(pltpu.SemaphoreType.DMA for _ in range(NBUF)),
            pltpu.SemaphoreType.DMA,
        ],
    )(x, table)


def kernel(x, encoding_weight):
    out = _pe_lookup(x, encoding_weight)
    return out.reshape(N, S, D)


# speedup vs baseline: 1.0098x; 1.0098x over previous
import jax
import jax.numpy as jnp
from jax import lax
from jax.experimental import pallas as pl
from jax.experimental.pallas import tpu as pltpu
from jax.experimental.pallas import tpu_sc as plsc

N = 256
S = 256
D = 256
NC = 2
NS = 16
NW = NC * NS
IB = 32
JB = 64
NJ = S // JB
IPC = 2
JC = IPC * JB
STEPS = IB // IPC
NBUF = 3
LANES = 16


def _sc_body(x_hbm, table_hbm, out_hbm, idx_v, idxf_v, adnd_v, bufs,
             gsems, ssems, asem):
    wid = lax.axis_index("s") * NC + lax.axis_index("c")
    i0 = (wid // NJ) * IB
    j0 = (wid % NJ) * JB

    pltpu.sync_copy(x_hbm.at[pl.ds(i0, IB)], idx_v)
    a_copy = pltpu.make_async_copy(x_hbm.at[pl.ds(j0, JB)], adnd_v, asem)
    a_copy.start()

    for i_sub in range(IB):
        a = [idx_v[i_sub, pl.ds(j0 + c * LANES, LANES)]
             for c in range(JB // LANES)]
        for c in range(JB // LANES):
            idxf_v[pl.ds(i_sub * JB + c * LANES, LANES)] = a[c]

    def gather(k):
        b = k % NBUF
        return pltpu.make_async_copy(
            table_hbm.at[idxf_v.at[pl.ds(JC * k, JC)]], bufs[b], gsems[b])

    def stores(k):
        b = k % NBUF
        return [pltpu.make_async_copy(
            bufs[b].at[pl.ds(h * JB, JB)],
            out_hbm.at[pl.ds((i0 + IPC * k + h) * S + j0, JB)], ssems[b])
            for h in range(IPC)]

    def add_chunk(k):
        buf = bufs[k % NBUF]

        def row_body(r, _):
            a = [adnd_v[r, pl.ds(c * LANES, LANES)].astype(jnp.float32)
                 for c in range(D // LANES)]
            for h in range(IPC):
                for c in range(D // LANES):
                    plsc.addupdate(
                        buf.at[h * JB + r, pl.ds(c * LANES, LANES)], a[c])
            return 0

        lax.fori_loop(0, JB, row_body, 0)

    for k in range(NBUF):
        gather(k).start()
    a_copy.wait()
    for k in range(STEPS):
        gather(k).wait()
        add_chunk(k)
        for st in stores(k):
            st.start()
        if k + NBUF < STEPS:
            for st in stores(k):
                st.wait()
            gather(k + NBUF).start()
    for k in range(STEPS - NBUF, STEPS):
        for st in stores(k):
            st.wait()


@jax.jit
def _pe_lookup(x, table):
    mesh = plsc.VectorSubcoreMesh(core_axis_name="c", subcore_axis_name="s")
    return pl.kernel(
        _sc_body,
        out_type=jax.ShapeDtypeStruct((N * S, D), jnp.float32),
        mesh=mesh,
        scratch_types=[
            pltpu.VMEM((IB, S), jnp.int32),
            pltpu.VMEM((IB * JB,), jnp.int32),
            pltpu.VMEM((JB, D), jnp.int32),
            tuple(pltpu.VMEM((JC, D), jnp.float32) for _ in range(NBUF)),
            tuple(pltpu.SemaphoreType.DMA for _ in range(NBUF)),
            tuple(pltpu.SemaphoreType.DMA for _ in range(NBUF)),
            pltpu.SemaphoreType.DMA,
        ],
    )(x, table)


def kernel(x, encoding_weight):
    out = _pe_lookup(x, encoding_weight)
    return out.reshape(N, S, D)


# speedup vs baseline: 1.0311x; 1.0211x over previous
import jax
import jax.numpy as jnp
from jax import lax
from jax.experimental import pallas as pl
from jax.experimental.pallas import tpu as pltpu
from jax.experimental.pallas import tpu_sc as plsc

N = 256
S = 256
D = 256
NC = 2
NS = 16
NW = NC * NS
IB = 32
JB = 64
NJ = S // JB
JC = JB
STEPS = IB
NBUF = 6
GROUP = 2
LANES = 16


def _sc_body(x_hbm, table_hbm, out_hbm, idx_v, adnd_v, bufs, gsems, ssems,
             asem):
    wid = lax.axis_index("s") * NC + lax.axis_index("c")
    i0 = (wid // NJ) * IB
    j0 = (wid % NJ) * JB

    pltpu.sync_copy(x_hbm.at[pl.ds(i0, IB)], idx_v)
    a_copy = pltpu.make_async_copy(x_hbm.at[pl.ds(j0, JB)], adnd_v, asem)
    a_copy.start()

    def gather(k):
        b = k % NBUF
        return pltpu.make_async_copy(
            table_hbm.at[idx_v.at[k, pl.ds(j0, JB)]], bufs[b], gsems[b])

    def store(k):
        b = k % NBUF
        return pltpu.make_async_copy(
            bufs[b], out_hbm.at[pl.ds((i0 + k) * S + j0, JC)], ssems[b])

    def add_group(ks):
        blist = [bufs[k % NBUF] for k in ks]

        def row_body(r, _):
            a = [adnd_v[r, pl.ds(c * LANES, LANES)].astype(jnp.float32)
                 for c in range(D // LANES)]
            for buf in blist:
                for c in range(D // LANES):
                    plsc.addupdate(buf.at[r, pl.ds(c * LANES, LANES)], a[c])
            return 0

        lax.fori_loop(0, JC, row_body, 0)

    for k in range(NBUF):
        gather(k).start()
    a_copy.wait()
    for g in range(STEPS // GROUP):
        ks = [GROUP * g + t for t in range(GROUP)]
        for k in ks:
            gather(k).wait()
        add_group(ks)
        for k in ks:
            store(k).start()
        for k in ks:
            if k + NBUF < STEPS:
                store(k).wait()
                gather(k + NBUF).start()
    for k in range(STEPS - NBUF, STEPS):
        store(k).wait()


@jax.jit
def _pe_lookup(x, table):
    mesh = plsc.VectorSubcoreMesh(core_axis_name="c", subcore_axis_name="s")
    return pl.kernel(
        _sc_body,
        out_type=jax.ShapeDtypeStruct((N * S, D), jnp.float32),
        mesh=mesh,
        scratch_types=[
            pltpu.VMEM((IB, S), jnp.int32),
            pltpu.VMEM((JB, D), jnp.int32),
            tuple(pltpu.VMEM((JC, D), jnp.float32) for _ in range(NBUF)),
            tuple(pltpu.SemaphoreType.DMA for _ in range(NBUF)),
            tuple(pltpu.SemaphoreType.DMA for _ in range(NBUF)),
            pltpu.SemaphoreType.DMA,
        ],
    )(x, table)


def kernel(x, encoding_weight):
    out = _pe_lookup(x, encoding_weight)
    return out.reshape(N, S, D)
